# baseline (speedup 1.0000x reference)
import jax
import jax.numpy as jnp
from jax.experimental import pallas as pl
from jax.experimental.pallas import tpu as pltpu

A_ = 5
C_ = 80
CIN = 768
CH = 1024
COUT = A_ * (5 + C_)
EPS = 1e-5
H = 32
W = 32
NPIX = H * W


def _body(x_ref, w1_ref, gm_ref, bt_ref, w2_ref, b2_ref, o_ref,
          y_ref, ssum_ref, ssq_ref, sc_ref, sh_ref):
    p = pl.program_id(0)
    b = pl.program_id(1)

    @pl.when(p == 0)
    def _conv1():
        x = x_ref[0]
        acc = None
        for dx in range(3):
            xd = x[:, dx:dx + W, :].reshape((H + 2) * W, CIN)
            xc = jnp.concatenate(
                [xd[dy * W:dy * W + NPIX, :] for dy in range(3)], axis=1)
            d = jnp.dot(xc, w1_ref[dx],
                        preferred_element_type=jnp.float32)
            acc = d if acc is None else acc + d
        ps = jnp.sum(acc, axis=0, keepdims=True)
        pq = jnp.sum(acc * acc, axis=0, keepdims=True)

        @pl.when(b == 0)
        def _init_stats():
            ssum_ref[...] = ps
            ssq_ref[...] = pq

        @pl.when(b != 0)
        def _acc_stats():
            ssum_ref[...] += ps
            ssq_ref[...] += pq

        y_ref[b] = acc.astype(jnp.bfloat16).reshape(H, W, CH)

    @pl.when(p == 1)
    def _conv2():
        @pl.when(b == 0)
        def _bn_affine():
            n = jnp.float32(pl.num_programs(1) * NPIX)
            mean = ssum_ref[...] / n
            var = ssq_ref[...] / n - mean * mean
            scale = gm_ref[...] * jax.lax.rsqrt(var + EPS)
            sc_ref[...] = scale
            sh_ref[...] = bt_ref[...] - mean * scale

        y = y_ref[b].reshape(NPIX, CH).astype(jnp.float32)
        z = y * sc_ref[...] + sh_ref[...]
        z = jnp.where(z > 0, z, 0.1 * z).astype(jnp.bfloat16)
        o = jnp.dot(z, w2_ref[...], preferred_element_type=jnp.float32)
        o_ref[0] = (o + b2_ref[...]).reshape(H, W, COUT)


def kernel(features, W1, gamma, beta, W2, b2):
    B = features.shape[0]
    xv = jnp.transpose(features, (0, 2, 3, 1))
    xv = jnp.pad(xv, ((0, 0), (1, 1), (1, 1), (0, 0))).astype(jnp.bfloat16)
    w1 = jnp.transpose(W1, (3, 2, 1, 0)).astype(jnp.bfloat16)
    w1 = w1.reshape(3, 3 * CIN, CH)
    w2 = jnp.transpose(W2[:, :, 0, 0]).astype(jnp.bfloat16)

    out = pl.pallas_call(
        _body,
        grid=(2, B),
        in_specs=[
            pl.BlockSpec((1, H + 2, W + 2, CIN),
                         lambda p, b: ((1 - p) * b, 0, 0, 0)),
            pl.BlockSpec((3, 3 * CIN, CH), lambda p, b: (0, 0, 0)),
            pl.BlockSpec((1, CH), lambda p, b: (0, 0)),
            pl.BlockSpec((1, CH), lambda p, b: (0, 0)),
            pl.BlockSpec((CH, COUT), lambda p, b: (0, 0)),
            pl.BlockSpec((1, COUT), lambda p, b: (0, 0)),
        ],
        out_specs=pl.BlockSpec((1, H, W, COUT), lambda p, b: (p * b, 0, 0, 0)),
        out_shape=jax.ShapeDtypeStruct((B, H, W, COUT), jnp.float32),
        scratch_shapes=[
            pltpu.VMEM((B, H, W, CH), jnp.bfloat16),
            pltpu.VMEM((1, CH), jnp.float32),
            pltpu.VMEM((1, CH), jnp.float32),
            pltpu.VMEM((1, CH), jnp.float32),
            pltpu.VMEM((1, CH), jnp.float32),
        ],
        compiler_params=pltpu.CompilerParams(
            dimension_semantics=("arbitrary", "arbitrary")),
    )(xv, w1, gamma.reshape(1, CH), beta.reshape(1, CH), w2,
      b2.reshape(1, COUT))

    return out


# speedup vs baseline: 1.1126x; 1.1126x over previous
import jax
import jax.numpy as jnp
from jax.experimental import pallas as pl
from jax.experimental.pallas import tpu as pltpu

A_ = 5
C_ = 80
CIN = 768
CH = 1024
COUT = A_ * (5 + C_)
EPS = 1e-5
H = 32
W = 32
NPIX = H * W


def _conv1_body(x_ref, w1_ref, y_ref, ps_ref, pq_ref):
    x = x_ref[0]
    acc = None
    for dx in range(3):
        xd = x[:, dx:dx + W, :].reshape((H + 2) * W, CIN)
        for dy in range(3):
            xm = xd[dy * W:dy * W + NPIX, :]
            d = jnp.dot(xm, w1_ref[dy, dx],
                        preferred_element_type=jnp.float32)
            acc = d if acc is None else acc + d
    ps_ref[0, 0] = jnp.sum(acc, axis=0)
    pq_ref[0, 0] = jnp.sum(acc * acc, axis=0)
    y_ref[0] = acc.astype(jnp.bfloat16).reshape(H, W, CH)


def _conv2_body(y_ref, ps_ref, pq_ref, gm_ref, bt_ref, w2_ref, b2_ref,
                o_ref):
    n = jnp.float32(ps_ref.shape[0] * NPIX)
    mean = jnp.sum(ps_ref[:, 0, :], axis=0, keepdims=True) / n
    var = jnp.sum(pq_ref[:, 0, :], axis=0, keepdims=True) / n - mean * mean
    scale = gm_ref[...] * jax.lax.rsqrt(var + EPS)
    shift = bt_ref[...] - mean * scale
    y = y_ref[0].reshape(NPIX, CH).astype(jnp.float32)
    z = y * scale + shift
    z = jnp.where(z > 0, z, 0.1 * z).astype(jnp.bfloat16)
    o = jnp.dot(z, w2_ref[...], preferred_element_type=jnp.float32)
    o_ref[0] = (o + b2_ref[...]).reshape(H, W, COUT)


def kernel(features, W1, gamma, beta, W2, b2):
    B = features.shape[0]
    xv = jnp.transpose(features, (0, 2, 3, 1))
    xv = jnp.pad(xv, ((0, 0), (1, 1), (1, 1), (0, 0))).astype(jnp.bfloat16)
    w1 = jnp.transpose(W1, (2, 3, 1, 0)).astype(jnp.bfloat16)
    w2 = jnp.transpose(W2[:, :, 0, 0]).astype(jnp.bfloat16)

    y, ps, pq = pl.pallas_call(
        _conv1_body,
        grid=(B,),
        in_specs=[
            pl.BlockSpec((1, H + 2, W + 2, CIN), lambda b: (b, 0, 0, 0)),
            pl.BlockSpec((3, 3, CIN, CH), lambda b: (0, 0, 0, 0)),
        ],
        out_specs=[
            pl.BlockSpec((1, H, W, CH), lambda b: (b, 0, 0, 0)),
            pl.BlockSpec((1, 1, CH), lambda b: (b, 0, 0)),
            pl.BlockSpec((1, 1, CH), lambda b: (b, 0, 0)),
        ],
        out_shape=[
            jax.ShapeDtypeStruct((B, H, W, CH), jnp.bfloat16),
            jax.ShapeDtypeStruct((B, 1, CH), jnp.float32),
            jax.ShapeDtypeStruct((B, 1, CH), jnp.float32),
        ],
        compiler_params=pltpu.CompilerParams(
            dimension_semantics=("parallel",)),
    )(xv, w1)

    out = pl.pallas_call(
        _conv2_body,
        grid=(B,),
        in_specs=[
            pl.BlockSpec((1, H, W, CH), lambda b: (b, 0, 0, 0)),
            pl.BlockSpec((B, 1, CH), lambda b: (0, 0, 0)),
            pl.BlockSpec((B, 1, CH), lambda b: (0, 0, 0)),
            pl.BlockSpec((1, CH), lambda b: (0, 0)),
            pl.BlockSpec((1, CH), lambda b: (0, 0)),
            pl.BlockSpec((CH, COUT), lambda b: (0, 0)),
            pl.BlockSpec((1, COUT), lambda b: (0, 0)),
        ],
        out_specs=pl.BlockSpec((1, H, W, COUT), lambda b: (b, 0, 0, 0)),
        out_shape=jax.ShapeDtypeStruct((B, H, W, COUT), jnp.float32),
        compiler_params=pltpu.CompilerParams(
            dimension_semantics=("parallel",)),
    )(y, ps, pq, gamma.reshape(1, CH), beta.reshape(1, CH), w2,
      b2.reshape(1, COUT))

    return out


# speedup vs baseline: 1.1309x; 1.0164x over previous
import jax
import jax.numpy as jnp
from jax.experimental import pallas as pl
from jax.experimental.pallas import tpu as pltpu

A_ = 5
C_ = 80
CIN = 768
CH = 1024
COUT = A_ * (5 + C_)
EPS = 1e-5
H = 32
W = 32
NPIX = H * W


def _body(x_ref, w1_ref, gm_ref, bt_ref, w2_ref, b2_ref, o_ref,
          y_ref, ssum_ref, ssq_ref, sc_ref, sh_ref):
    p = pl.program_id(0)
    b = pl.program_id(1)

    @pl.when(p == 0)
    def _conv1():
        x = x_ref[0]
        acc = None
        for dx in range(3):
            xd = x[:, dx:dx + W, :].reshape((H + 2) * W, CIN)
            for dy in range(3):
                xm = xd[dy * W:dy * W + NPIX, :]
                d = jnp.dot(xm, w1_ref[dy, dx],
                            preferred_element_type=jnp.float32)
                acc = d if acc is None else acc + d
        ps = jnp.sum(acc, axis=0, keepdims=True)
        pq = jnp.sum(acc * acc, axis=0, keepdims=True)

        @pl.when(b == 0)
        def _init_stats():
            ssum_ref[...] = ps
            ssq_ref[...] = pq

        @pl.when(b != 0)
        def _acc_stats():
            ssum_ref[...] += ps
            ssq_ref[...] += pq

        y_ref[b] = acc.astype(jnp.bfloat16).reshape(H, W, CH)

    @pl.when(p == 1)
    def _conv2():
        @pl.when(b == 0)
        def _bn_affine():
            n = jnp.float32(pl.num_programs(1) * NPIX)
            mean = ssum_ref[...] / n
            var = ssq_ref[...] / n - mean * mean
            scale = gm_ref[...] * jax.lax.rsqrt(var + EPS)
            sc_ref[...] = scale
            sh_ref[...] = bt_ref[...] - mean * scale

        y = y_ref[b].reshape(NPIX, CH).astype(jnp.float32)
        z = y * sc_ref[...] + sh_ref[...]
        z = jnp.where(z > 0, z, 0.1 * z).astype(jnp.bfloat16)
        o = jnp.dot(z, w2_ref[...], preferred_element_type=jnp.float32)
        o_ref[0] = (o + b2_ref[...]).reshape(H, W, COUT)


def kernel(features, W1, gamma, beta, W2, b2):
    B = features.shape[0]
    xv = jnp.transpose(features, (0, 2, 3, 1))
    xv = jnp.pad(xv, ((0, 0), (1, 1), (1, 1), (0, 0))).astype(jnp.bfloat16)
    w1 = jnp.transpose(W1, (2, 3, 1, 0)).astype(jnp.bfloat16)
    w2 = jnp.transpose(W2[:, :, 0, 0]).astype(jnp.bfloat16)

    out = pl.pallas_call(
        _body,
        grid=(2, B),
        in_specs=[
            pl.BlockSpec((1, H + 2, W + 2, CIN),
                         lambda p, b: ((1 - p) * b, 0, 0, 0)),
            pl.BlockSpec((3, 3, CIN, CH), lambda p, b: (0, 0, 0, 0)),
            pl.BlockSpec((1, CH), lambda p, b: (0, 0)),
            pl.BlockSpec((1, CH), lambda p, b: (0, 0)),
            pl.BlockSpec((CH, COUT), lambda p, b: (0, 0)),
            pl.BlockSpec((1, COUT), lambda p, b: (0, 0)),
        ],
        out_specs=pl.BlockSpec((1, H, W, COUT), lambda p, b: (p * b, 0, 0, 0)),
        out_shape=jax.ShapeDtypeStruct((B, H, W, COUT), jnp.float32),
        scratch_shapes=[
            pltpu.VMEM((B, H, W, CH), jnp.bfloat16),
            pltpu.VMEM((1, CH), jnp.float32),
            pltpu.VMEM((1, CH), jnp.float32),
            pltpu.VMEM((1, CH), jnp.float32),
            pltpu.VMEM((1, CH), jnp.float32),
        ],
        compiler_params=pltpu.CompilerParams(
            dimension_semantics=("arbitrary", "arbitrary")),
    )(xv, w1, gamma.reshape(1, CH), beta.reshape(1, CH), w2,
      b2.reshape(1, COUT))

    return out
